# Initial kernel scaffold; baseline (speedup 1.0000x reference)
import jax
import jax.numpy as jnp
from jax.experimental import pallas as pl


def kernel(x, edge_index, W0, W1):
    raise NotImplementedError("write your pallas kernel here")


# speedup vs baseline: 24.2644x; 24.2644x over previous
import functools

import jax
import jax.numpy as jnp
from jax import lax
from jax.experimental import pallas as pl
from jax.experimental.pallas import tpu as pltpu
from jax.experimental.pallas import tpu_sc as plsc

N_NODES = 10000
N_PAD = 10240
D_HID = 16
E_EDGES = 320000
NW = 32
CHUNK = 128
EPT = 10240
NCHUNK = EPT // CHUNK
ROWS_PT = N_PAD // 16

_SC_MESH = plsc.VectorSubcoreMesh(core_axis_name="c", subcore_axis_name="s")
_SC_PARAMS = pltpu.CompilerParams(use_tc_tiling_on_sc=False)


@functools.partial(
    pl.kernel,
    out_type=jax.ShapeDtypeStruct((2, N_PAD), jnp.float32),
    mesh=_SC_MESH,
    scratch_types=[
        pltpu.VMEM((NCHUNK, CHUNK), jnp.int32),
        pltpu.VMEM((CHUNK,), jnp.float32),
        pltpu.VMEM_SHARED((N_PAD,), jnp.float32),
    ],
    compiler_params=_SC_PARAMS,
)
def _deg_kernel(dst2d, ones_init, zeros_init, out, didx, ones_v, dacc):
    cid = lax.axis_index("c")
    sid = lax.axis_index("s")
    w = cid * 16 + sid
    pltpu.sync_copy(dst2d.at[pl.ds(w * NCHUNK, NCHUNK)], didx)
    pltpu.sync_copy(ones_init.at[pl.ds(0, CHUNK)], ones_v)
    row0 = sid * ROWS_PT

    @pl.when(cid == 0)
    def _():
        pltpu.sync_copy(ones_init.at[pl.ds(row0, ROWS_PT)],
                        dacc.at[pl.ds(row0, ROWS_PT)])

    @pl.when(cid == 1)
    def _():
        pltpu.sync_copy(zeros_init.at[pl.ds(row0, ROWS_PT)],
                        dacc.at[pl.ds(row0, ROWS_PT)])

    plsc.subcore_barrier()

    def body(j, carry):
        pltpu.sync_copy(ones_v, dacc.at[didx.at[j]], add=True)
        return carry

    lax.fori_loop(0, NCHUNK, body, 0)
    plsc.subcore_barrier()
    pltpu.sync_copy(dacc.at[pl.ds(row0, ROWS_PT)],
                    out.at[cid, pl.ds(row0, ROWS_PT)])


@functools.partial(
    pl.kernel,
    out_type=jax.ShapeDtypeStruct((2, N_PAD, D_HID), jnp.float32),
    mesh=_SC_MESH,
    scratch_types=[
        pltpu.VMEM((NCHUNK, CHUNK), jnp.int32),
        pltpu.VMEM((NCHUNK, CHUNK), jnp.int32),
        pltpu.VMEM((CHUNK, D_HID), jnp.float32),
        pltpu.VMEM_SHARED((N_PAD, D_HID), jnp.float32),
    ],
    compiler_params=_SC_PARAMS,
)
def _agg_kernel(src2d, dst2d, hp, zinit, out, sidx, didx, buf, acc):
    cid = lax.axis_index("c")
    sid = lax.axis_index("s")
    w = cid * 16 + sid
    pltpu.sync_copy(src2d.at[pl.ds(w * NCHUNK, NCHUNK)], sidx)
    pltpu.sync_copy(dst2d.at[pl.ds(w * NCHUNK, NCHUNK)], didx)
    row0 = sid * ROWS_PT

    @pl.when(cid == 0)
    def _():
        pltpu.sync_copy(hp.at[pl.ds(row0, ROWS_PT)],
                        acc.at[pl.ds(row0, ROWS_PT)])

    @pl.when(cid == 1)
    def _():
        pltpu.sync_copy(zinit.at[pl.ds(row0, ROWS_PT)],
                        acc.at[pl.ds(row0, ROWS_PT)])

    plsc.subcore_barrier()

    def body(j, carry):
        pltpu.sync_copy(hp.at[sidx.at[j]], buf)
        pltpu.sync_copy(buf, acc.at[didx.at[j]], add=True)
        return carry

    lax.fori_loop(0, NCHUNK, body, 0)
    plsc.subcore_barrier()
    pltpu.sync_copy(acc.at[pl.ds(row0, ROWS_PT)],
                    out.at[cid, pl.ds(row0, ROWS_PT)])


def _mm_body(x_ref, w_ref, o_ref):
    o_ref[...] = jnp.dot(x_ref[...], w_ref[...],
                         preferred_element_type=jnp.float32)


def _scale_body(h0_ref, d0_ref, d1_ref, dis_ref, hp_ref):
    dis = lax.rsqrt(d0_ref[...] + d1_ref[...])
    dis_ref[...] = dis
    hp_ref[...] = h0_ref[...] * dis


def _relu_scale_body(p_ref, dis_ref, g1_ref):
    dis = dis_ref[...]
    g1_ref[...] = dis * jnp.maximum(dis * (p_ref[0] + p_ref[1]), 0.0)


def _final_body(q_ref, dis_ref, w1_ref, o_ref):
    q = dis_ref[...] * (q_ref[0] + q_ref[1])
    o_ref[...] = jnp.dot(q, w1_ref[...], preferred_element_type=jnp.float32)


def kernel(x, edge_index, W0, W1):
    f32 = jnp.float32
    src = edge_index[0]
    dst = edge_index[1]

    pad = NW * EPT - E_EDGES
    pad_src = jnp.zeros((pad,), jnp.int32)
    pad_dst = (N_NODES + (jnp.arange(pad, dtype=jnp.int32) % 16))
    src2d = jnp.concatenate([src, pad_src]).reshape(NW * NCHUNK, CHUNK)
    dst2d = jnp.concatenate([dst, pad_dst]).reshape(NW * NCHUNK, CHUNK)
    x_pad = jnp.pad(x, ((0, N_PAD - N_NODES), (0, 0)))
    ones1 = jnp.ones((N_PAD,), f32)
    zeros1 = jnp.zeros((N_PAD,), f32)
    zeros16 = jnp.zeros((N_PAD, D_HID), f32)

    h0 = pl.pallas_call(
        _mm_body,
        grid=(8,),
        in_specs=[
            pl.BlockSpec((N_PAD // 8, 128), lambda i: (i, 0)),
            pl.BlockSpec((128, D_HID), lambda i: (0, 0)),
        ],
        out_specs=pl.BlockSpec((N_PAD // 8, D_HID), lambda i: (i, 0)),
        out_shape=jax.ShapeDtypeStruct((N_PAD, D_HID), f32),
    )(x_pad, W0)

    deg = _deg_kernel(dst2d, ones1, zeros1)

    d0 = deg[0].reshape(N_PAD, 1)
    d1 = deg[1].reshape(N_PAD, 1)
    dis, hp = pl.pallas_call(
        _scale_body,
        out_shape=(
            jax.ShapeDtypeStruct((N_PAD, 1), f32),
            jax.ShapeDtypeStruct((N_PAD, D_HID), f32),
        ),
    )(h0, d0, d1)

    p = _agg_kernel(src2d, dst2d, hp, zeros16)

    g1 = pl.pallas_call(
        _relu_scale_body,
        out_shape=jax.ShapeDtypeStruct((N_PAD, D_HID), f32),
    )(p, dis)

    q = _agg_kernel(src2d, dst2d, g1, zeros16)

    out = pl.pallas_call(
        _final_body,
        out_shape=jax.ShapeDtypeStruct((N_PAD, 7), f32),
    )(q, dis, W1)

    return out[:N_NODES]


# speedup vs baseline: 31.5329x; 1.2996x over previous
import functools

import jax
import jax.numpy as jnp
from jax import lax
from jax.experimental import pallas as pl
from jax.experimental.pallas import tpu as pltpu
from jax.experimental.pallas import tpu_sc as plsc

N_NODES = 10000
N_PAD = 10240
D_HID = 16
E_EDGES = 320000
NW = 32
CHUNK = 128
EPT = 10240
NCHUNK = EPT // CHUNK
ROWS_PT = N_PAD // 16
NBUF = 8

_SC_MESH = plsc.VectorSubcoreMesh(core_axis_name="c", subcore_axis_name="s")
_SC_PARAMS = pltpu.CompilerParams(use_tc_tiling_on_sc=False)


@functools.partial(
    pl.kernel,
    out_type=jax.ShapeDtypeStruct((2, N_PAD), jnp.float32),
    mesh=_SC_MESH,
    scratch_types=[
        pltpu.VMEM((NCHUNK, CHUNK), jnp.int32),
        pltpu.VMEM((CHUNK,), jnp.float32),
        [pltpu.SemaphoreType.DMA for _ in range(NBUF)],
        pltpu.VMEM_SHARED((N_PAD,), jnp.float32),
    ],
    compiler_params=_SC_PARAMS,
)
def _deg_kernel(dst2d, ones_init, zeros_init, out, didx, ones_v, ssems, dacc):
    cid = lax.axis_index("c")
    sid = lax.axis_index("s")
    w = cid * 16 + sid
    pltpu.sync_copy(dst2d.at[pl.ds(w * NCHUNK, NCHUNK)], didx)
    pltpu.sync_copy(ones_init.at[pl.ds(0, CHUNK)], ones_v)
    row0 = sid * ROWS_PT

    @pl.when(cid == 0)
    def _():
        pltpu.sync_copy(ones_init.at[pl.ds(row0, ROWS_PT)],
                        dacc.at[pl.ds(row0, ROWS_PT)])

    @pl.when(cid == 1)
    def _():
        pltpu.sync_copy(zeros_init.at[pl.ds(row0, ROWS_PT)],
                        dacc.at[pl.ds(row0, ROWS_PT)])

    plsc.subcore_barrier()

    def body(i, carry):
        jj = i * NBUF
        sds = [
            pltpu.async_copy(ones_v, dacc.at[didx.at[jj + b]], ssems[b],
                             add=True)
            for b in range(NBUF)
        ]
        for d in sds:
            d.wait()
        return carry

    lax.fori_loop(0, NCHUNK // NBUF, body, 0)
    plsc.subcore_barrier()
    pltpu.sync_copy(dacc.at[pl.ds(row0, ROWS_PT)],
                    out.at[cid, pl.ds(row0, ROWS_PT)])


@functools.partial(
    pl.kernel,
    out_type=jax.ShapeDtypeStruct((2, N_PAD, D_HID), jnp.float32),
    mesh=_SC_MESH,
    scratch_types=[
        pltpu.VMEM((NCHUNK, CHUNK), jnp.int32),
        pltpu.VMEM((NCHUNK, CHUNK), jnp.int32),
        [pltpu.VMEM((CHUNK, D_HID), jnp.float32) for _ in range(NBUF)],
        [pltpu.SemaphoreType.DMA for _ in range(NBUF)],
        [pltpu.SemaphoreType.DMA for _ in range(NBUF)],
        pltpu.VMEM_SHARED((N_PAD, D_HID), jnp.float32),
    ],
    compiler_params=_SC_PARAMS,
)
def _agg_kernel(src2d, dst2d, hp, zinit, out, sidx, didx, bufs, gsems, ssems,
                acc):
    cid = lax.axis_index("c")
    sid = lax.axis_index("s")
    w = cid * 16 + sid
    pltpu.sync_copy(src2d.at[pl.ds(w * NCHUNK, NCHUNK)], sidx)
    pltpu.sync_copy(dst2d.at[pl.ds(w * NCHUNK, NCHUNK)], didx)
    row0 = sid * ROWS_PT

    @pl.when(cid == 0)
    def _():
        pltpu.sync_copy(hp.at[pl.ds(row0, ROWS_PT)],
                        acc.at[pl.ds(row0, ROWS_PT)])

    @pl.when(cid == 1)
    def _():
        pltpu.sync_copy(zinit.at[pl.ds(row0, ROWS_PT)],
                        acc.at[pl.ds(row0, ROWS_PT)])

    plsc.subcore_barrier()

    def body(i, carry):
        jj = i * NBUF
        gds = [
            pltpu.async_copy(hp.at[sidx.at[jj + b]], bufs[b], gsems[b])
            for b in range(NBUF)
        ]
        sds = []
        for b in range(NBUF):
            gds[b].wait()
            sds.append(
                pltpu.async_copy(bufs[b], acc.at[didx.at[jj + b]], ssems[b],
                                 add=True))
        for d in sds:
            d.wait()
        return carry

    lax.fori_loop(0, NCHUNK // NBUF, body, 0)
    plsc.subcore_barrier()
    pltpu.sync_copy(acc.at[pl.ds(row0, ROWS_PT)],
                    out.at[cid, pl.ds(row0, ROWS_PT)])


def _mm_body(x_ref, w_ref, o_ref):
    o_ref[...] = jnp.dot(x_ref[...], w_ref[...],
                         preferred_element_type=jnp.float32)


def _scale_body(h0_ref, d0_ref, d1_ref, dis_ref, hp_ref):
    dis = lax.rsqrt(d0_ref[...] + d1_ref[...])
    dis_ref[...] = dis
    hp_ref[...] = h0_ref[...] * dis


def _relu_scale_body(p_ref, dis_ref, g1_ref):
    dis = dis_ref[...]
    g1_ref[...] = dis * jnp.maximum(dis * (p_ref[0] + p_ref[1]), 0.0)


def _final_body(q_ref, dis_ref, w1_ref, o_ref):
    q = dis_ref[...] * (q_ref[0] + q_ref[1])
    o_ref[...] = jnp.dot(q, w1_ref[...], preferred_element_type=jnp.float32)


def kernel(x, edge_index, W0, W1):
    f32 = jnp.float32
    src = edge_index[0]
    dst = edge_index[1]

    pad = NW * EPT - E_EDGES
    pad_src = jnp.zeros((pad,), jnp.int32)
    pad_dst = (N_NODES + (jnp.arange(pad, dtype=jnp.int32) % 16))
    src2d = jnp.concatenate([src, pad_src]).reshape(NW * NCHUNK, CHUNK)
    dst2d = jnp.concatenate([dst, pad_dst]).reshape(NW * NCHUNK, CHUNK)
    x_pad = jnp.pad(x, ((0, N_PAD - N_NODES), (0, 0)))
    ones1 = jnp.ones((N_PAD,), f32)
    zeros1 = jnp.zeros((N_PAD,), f32)
    zeros16 = jnp.zeros((N_PAD, D_HID), f32)

    h0 = pl.pallas_call(
        _mm_body,
        grid=(8,),
        in_specs=[
            pl.BlockSpec((N_PAD // 8, 128), lambda i: (i, 0)),
            pl.BlockSpec((128, D_HID), lambda i: (0, 0)),
        ],
        out_specs=pl.BlockSpec((N_PAD // 8, D_HID), lambda i: (i, 0)),
        out_shape=jax.ShapeDtypeStruct((N_PAD, D_HID), f32),
    )(x_pad, W0)

    deg = _deg_kernel(dst2d, ones1, zeros1)

    d0 = deg[0].reshape(N_PAD, 1)
    d1 = deg[1].reshape(N_PAD, 1)
    dis, hp = pl.pallas_call(
        _scale_body,
        out_shape=(
            jax.ShapeDtypeStruct((N_PAD, 1), f32),
            jax.ShapeDtypeStruct((N_PAD, D_HID), f32),
        ),
    )(h0, d0, d1)

    p = _agg_kernel(src2d, dst2d, hp, zeros16)

    g1 = pl.pallas_call(
        _relu_scale_body,
        out_shape=jax.ShapeDtypeStruct((N_PAD, D_HID), f32),
    )(p, dis)

    q = _agg_kernel(src2d, dst2d, g1, zeros16)

    out = pl.pallas_call(
        _final_body,
        out_shape=jax.ShapeDtypeStruct((N_PAD, 7), f32),
    )(q, dis, W1)

    return out[:N_NODES]


# speedup vs baseline: 48.2340x; 1.5296x over previous
import functools

import jax
import jax.numpy as jnp
from jax import lax
from jax.experimental import pallas as pl
from jax.experimental.pallas import tpu as pltpu
from jax.experimental.pallas import tpu_sc as plsc

N_NODES = 10000
N_PAD = 10240
D_HID = 16
E_EDGES = 320000
NW = 32
CHUNK = 128
EPT = 10240
NCHUNK = EPT // CHUNK
ROWS_PT = N_PAD // 16
NBUF = 8

_SC_MESH = plsc.VectorSubcoreMesh(core_axis_name="c", subcore_axis_name="s")
_SC_PARAMS = pltpu.CompilerParams(use_tc_tiling_on_sc=False)


@functools.partial(
    pl.kernel,
    out_type=jax.ShapeDtypeStruct((2, N_PAD), jnp.float32),
    mesh=_SC_MESH,
    scratch_types=[
        pltpu.VMEM((NCHUNK, CHUNK), jnp.int32),
        pltpu.VMEM((CHUNK,), jnp.float32),
        [pltpu.SemaphoreType.DMA for _ in range(NBUF)],
        pltpu.VMEM_SHARED((N_PAD,), jnp.float32),
    ],
    compiler_params=_SC_PARAMS,
)
def _deg_kernel(dst2d, ones_init, zeros_init, out, didx, ones_v, ssems, dacc):
    cid = lax.axis_index("c")
    sid = lax.axis_index("s")
    w = cid * 16 + sid
    pltpu.sync_copy(dst2d.at[pl.ds(w * NCHUNK, NCHUNK)], didx)
    pltpu.sync_copy(ones_init.at[pl.ds(0, CHUNK)], ones_v)
    row0 = sid * ROWS_PT

    @pl.when(cid == 0)
    def _():
        pltpu.sync_copy(ones_init.at[pl.ds(row0, ROWS_PT)],
                        dacc.at[pl.ds(row0, ROWS_PT)])

    @pl.when(cid == 1)
    def _():
        pltpu.sync_copy(zeros_init.at[pl.ds(row0, ROWS_PT)],
                        dacc.at[pl.ds(row0, ROWS_PT)])

    plsc.subcore_barrier()

    def body(i, carry):
        jj = i * NBUF
        sds = [
            pltpu.async_copy(ones_v, dacc.at[didx.at[jj + b]], ssems[b],
                             add=True)
            for b in range(NBUF)
        ]
        for d in sds:
            d.wait()
        return carry

    lax.fori_loop(0, NCHUNK // NBUF, body, 0)
    plsc.subcore_barrier()
    pltpu.sync_copy(dacc.at[pl.ds(row0, ROWS_PT)],
                    out.at[cid, pl.ds(row0, ROWS_PT)])


@functools.partial(
    pl.kernel,
    out_type=jax.ShapeDtypeStruct((2, N_PAD, D_HID), jnp.float32),
    mesh=_SC_MESH,
    scratch_types=[
        pltpu.VMEM((NCHUNK, CHUNK), jnp.int32),
        pltpu.VMEM((NCHUNK, CHUNK), jnp.int32),
        [pltpu.VMEM((CHUNK, D_HID), jnp.float32) for _ in range(NBUF)],
        [pltpu.SemaphoreType.DMA for _ in range(NBUF)],
        [pltpu.SemaphoreType.DMA for _ in range(NBUF)],
        pltpu.VMEM_SHARED((N_PAD, D_HID), jnp.float32),
        pltpu.VMEM_SHARED((N_PAD, D_HID), jnp.float32),
    ],
    compiler_params=_SC_PARAMS,
)
def _agg_kernel(src2d, dst2d, hp, zinit, out, sidx, didx, bufs, gsems, ssems,
                acc, tbl):
    cid = lax.axis_index("c")
    sid = lax.axis_index("s")
    w = cid * 16 + sid
    pltpu.sync_copy(src2d.at[pl.ds(w * NCHUNK, NCHUNK)], sidx)
    pltpu.sync_copy(dst2d.at[pl.ds(w * NCHUNK, NCHUNK)], didx)
    row0 = sid * ROWS_PT

    pltpu.sync_copy(hp.at[pl.ds(row0, ROWS_PT)], tbl.at[pl.ds(row0, ROWS_PT)])
    pltpu.sync_copy(zinit.at[pl.ds(row0, ROWS_PT)],
                    acc.at[pl.ds(row0, ROWS_PT)])

    plsc.subcore_barrier()

    def body(i, carry):
        jj = i * NBUF
        gds = [
            pltpu.async_copy(tbl.at[sidx.at[jj + b]], bufs[b], gsems[b])
            for b in range(NBUF)
        ]
        sds = []
        for b in range(NBUF):
            gds[b].wait()
            sds.append(
                pltpu.async_copy(bufs[b], acc.at[didx.at[jj + b]], ssems[b],
                                 add=True))
        for d in sds:
            d.wait()
        return carry

    lax.fori_loop(0, NCHUNK // NBUF, body, 0)
    plsc.subcore_barrier()
    pltpu.sync_copy(acc.at[pl.ds(row0, ROWS_PT)],
                    out.at[cid, pl.ds(row0, ROWS_PT)])


def _mm_body(x_ref, w_ref, o_ref):
    o_ref[...] = jnp.dot(x_ref[...], w_ref[...],
                         preferred_element_type=jnp.float32)


def _scale_body(h0_ref, d0_ref, d1_ref, dis_ref, hp_ref):
    dis = lax.rsqrt(d0_ref[...] + d1_ref[...])
    dis_ref[...] = dis
    hp_ref[...] = h0_ref[...] * dis


def _relu_scale_body(p_ref, hp_ref, dis_ref, g1_ref):
    dis = dis_ref[...]
    g1_ref[...] = dis * jnp.maximum(
        dis * (p_ref[0] + p_ref[1] + hp_ref[...]), 0.0)


def _final_body(q_ref, g1_ref, dis_ref, w1_ref, o_ref):
    q = dis_ref[...] * (q_ref[0] + q_ref[1] + g1_ref[...])
    o_ref[...] = jnp.dot(q, w1_ref[...], preferred_element_type=jnp.float32)


def kernel(x, edge_index, W0, W1):
    f32 = jnp.float32
    src = edge_index[0]
    dst = edge_index[1]

    pad = NW * EPT - E_EDGES
    pad_src = jnp.zeros((pad,), jnp.int32)
    pad_dst = (N_NODES + (jnp.arange(pad, dtype=jnp.int32) % 16))
    src2d = jnp.concatenate([src, pad_src]).reshape(NW * NCHUNK, CHUNK)
    dst2d = jnp.concatenate([dst, pad_dst]).reshape(NW * NCHUNK, CHUNK)
    x_pad = jnp.pad(x, ((0, N_PAD - N_NODES), (0, 0)))
    ones1 = jnp.ones((N_PAD,), f32)
    zeros1 = jnp.zeros((N_PAD,), f32)
    zeros16 = jnp.zeros((N_PAD, D_HID), f32)

    h0 = pl.pallas_call(
        _mm_body,
        grid=(8,),
        in_specs=[
            pl.BlockSpec((N_PAD // 8, 128), lambda i: (i, 0)),
            pl.BlockSpec((128, D_HID), lambda i: (0, 0)),
        ],
        out_specs=pl.BlockSpec((N_PAD // 8, D_HID), lambda i: (i, 0)),
        out_shape=jax.ShapeDtypeStruct((N_PAD, D_HID), f32),
    )(x_pad, W0)

    deg = _deg_kernel(dst2d, ones1, zeros1)

    d0 = deg[0].reshape(N_PAD, 1)
    d1 = deg[1].reshape(N_PAD, 1)
    dis, hp = pl.pallas_call(
        _scale_body,
        out_shape=(
            jax.ShapeDtypeStruct((N_PAD, 1), f32),
            jax.ShapeDtypeStruct((N_PAD, D_HID), f32),
        ),
    )(h0, d0, d1)

    p = _agg_kernel(src2d, dst2d, hp, zeros16)

    g1 = pl.pallas_call(
        _relu_scale_body,
        out_shape=jax.ShapeDtypeStruct((N_PAD, D_HID), f32),
    )(p, hp, dis)

    q = _agg_kernel(src2d, dst2d, g1, zeros16)

    out = pl.pallas_call(
        _final_body,
        out_shape=jax.ShapeDtypeStruct((N_PAD, 7), f32),
    )(q, g1, dis, W1)

    return out[:N_NODES]
